# baseline (speedup 1.0000x reference)
import functools

import jax
import jax.numpy as jnp
from jax import lax
from jax.experimental import pallas as pl
from jax.experimental.pallas import tpu as pltpu
from jax.experimental.pallas import tpu_sc as plsc

N_NODES_ = 10000
N_EDGES_ = 320000
NC = 2
NS = 16
NW = NC * NS
BLK = 128
NBLK = 80
EPW = NBLK * BLK
E_PAD = NW * EPW
K_PAD = E_PAD - N_EDGES_
NBUF = 5
NGRP = NBLK // NBUF
DEG_CHUNK = 624
HALF = DEG_CHUNK // 2

_MESH = plsc.VectorSubcoreMesh(
    core_axis_name="c", subcore_axis_name="s", num_cores=NC, num_subcores=NS)


@functools.partial(
    pl.kernel,
    out_type=jax.ShapeDtypeStruct((NC * N_NODES_,), jnp.float32),
    mesh=_MESH,
    scratch_types=[
        pltpu.VMEM((NBLK, BLK), jnp.int32),
        pltpu.VMEM((BLK,), jnp.float32),
        pltpu.VMEM((DEG_CHUNK + 16,), jnp.float32),
        pltpu.VMEM_SHARED((N_NODES_,), jnp.float32),
        pltpu.SemaphoreType.DMA,
    ],
    compiler_params=pltpu.CompilerParams(use_tc_tiling_on_sc=False),
)
def _deg_kernel(ei_hbm, out_hbm, dst_v, ones_v, zero_v, acc, sem):
    cid = lax.axis_index("c")
    sid = lax.axis_index("s")
    wid = sid * NC + cid

    one16 = jnp.full((16,), 1.0, jnp.float32)
    zero16 = jnp.zeros((16,), jnp.float32)
    for i in range(BLK // 16):
        ones_v[pl.ds(i * 16, 16)] = one16
    for i in range((DEG_CHUNK + 16) // 16):
        zero_v[pl.ds(i * 16, 16)] = zero16

    pltpu.sync_copy(ei_hbm.at[1, pl.ds(wid * NBLK, NBLK)], dst_v)

    pltpu.sync_copy(zero_v.at[pl.ds(0, DEG_CHUNK)],
                    acc.at[pl.ds(sid * DEG_CHUNK, DEG_CHUNK)])

    @pl.when(sid == NS - 1)
    def _():
        pltpu.sync_copy(zero_v.at[pl.ds(DEG_CHUNK, 16)],
                        acc.at[pl.ds(NS * DEG_CHUNK, 16)])

    plsc.subcore_barrier()

    def body(i, carry):
        pltpu.async_copy(ones_v, acc.at[dst_v.at[i]], sem, add=True)
        return carry

    lax.fori_loop(0, NBLK, body, 0)

    def drain(i, carry):
        pltpu.make_async_copy(out_hbm.at[pl.ds(0, BLK)], ones_v, sem).wait()
        return carry

    lax.fori_loop(0, NBLK, drain, 0)
    plsc.subcore_barrier()

    obase = cid * N_NODES_
    pltpu.sync_copy(acc.at[pl.ds(sid * DEG_CHUNK, DEG_CHUNK)],
                    zero_v.at[pl.ds(0, DEG_CHUNK)])
    pltpu.sync_copy(zero_v.at[pl.ds(0, DEG_CHUNK)],
                    out_hbm.at[pl.ds(obase + sid * DEG_CHUNK, DEG_CHUNK)])

    @pl.when(sid == NS - 1)
    def _():
        pltpu.sync_copy(acc.at[pl.ds(NS * DEG_CHUNK, 16)],
                        zero_v.at[pl.ds(DEG_CHUNK, 16)])
        pltpu.sync_copy(zero_v.at[pl.ds(DEG_CHUNK, 16)],
                        out_hbm.at[pl.ds(obase + NS * DEG_CHUNK, 16)])


def _make_scatter(F):
    @functools.partial(
        pl.kernel,
        out_type=jax.ShapeDtypeStruct((NC, N_NODES_, F), jnp.float32),
        mesh=_MESH,
        scratch_types=[
            pltpu.VMEM((NBLK, BLK), jnp.int32),
            pltpu.VMEM((NBLK, BLK), jnp.int32),
            pltpu.VMEM((NBUF, BLK, F), jnp.float32),
            pltpu.VMEM((HALF + 16, F), jnp.float32),
            pltpu.VMEM_SHARED((N_NODES_, F), jnp.float32),
            pltpu.SemaphoreType.DMA((NBUF,)),
            pltpu.SemaphoreType.DMA((NBUF,)),
        ],
        compiler_params=pltpu.CompilerParams(use_tc_tiling_on_sc=False),
    )
    def _scatter_kernel(y_hbm, ei_hbm, out_hbm,
                        src_v, dst_v, rows_v, stage_v, acc, gsem, ssem):
        cid = lax.axis_index("c")
        sid = lax.axis_index("s")
        wid = sid * NC + cid

        pltpu.sync_copy(ei_hbm.at[0, pl.ds(wid * NBLK, NBLK)], src_v)
        pltpu.sync_copy(ei_hbm.at[1, pl.ds(wid * NBLK, NBLK)], dst_v)

        for b in range(NBUF):
            pltpu.async_copy(y_hbm.at[src_v.at[b]], rows_v.at[b], gsem.at[b])

        r0 = sid * DEG_CHUNK
        for k in range(2):
            pltpu.sync_copy(y_hbm.at[pl.ds(r0 + k * HALF, HALF)],
                            stage_v.at[pl.ds(0, HALF)])
            pltpu.sync_copy(stage_v.at[pl.ds(0, HALF)],
                            acc.at[pl.ds(r0 + k * HALF, HALF)])

        @pl.when(sid == NS - 1)
        def _():
            pltpu.sync_copy(y_hbm.at[pl.ds(NS * DEG_CHUNK, 16)],
                            stage_v.at[pl.ds(HALF, 16)])
            pltpu.sync_copy(stage_v.at[pl.ds(HALF, 16)],
                            acc.at[pl.ds(NS * DEG_CHUNK, 16)])

        plsc.subcore_barrier()

        def outer(g, carry):
            for b in range(NBUF):
                pltpu.make_async_copy(y_hbm.at[pl.ds(0, BLK)],
                                      rows_v.at[b], gsem.at[b]).wait()
                pltpu.async_copy(rows_v.at[b], acc.at[dst_v.at[g * NBUF + b]],
                                 ssem.at[b], add=True)
            for b in range(NBUF):
                pltpu.make_async_copy(y_hbm.at[pl.ds(0, BLK)],
                                      rows_v.at[b], ssem.at[b]).wait()

                @pl.when(g + 1 < NGRP)
                def _():
                    pltpu.async_copy(y_hbm.at[src_v.at[(g + 1) * NBUF + b]],
                                     rows_v.at[b], gsem.at[b])

            return carry

        lax.fori_loop(0, NGRP, outer, 0)
        plsc.subcore_barrier()

        for k in range(2):
            pltpu.sync_copy(acc.at[pl.ds(r0 + k * HALF, HALF)],
                            stage_v.at[pl.ds(0, HALF)])
            pltpu.sync_copy(stage_v.at[pl.ds(0, HALF)],
                            out_hbm.at[cid, pl.ds(r0 + k * HALF, HALF)])

        @pl.when(sid == NS - 1)
        def _():
            pltpu.sync_copy(acc.at[pl.ds(NS * DEG_CHUNK, 16)],
                            stage_v.at[pl.ds(HALF, 16)])
            pltpu.sync_copy(stage_v.at[pl.ds(HALF, 16)],
                            out_hbm.at[cid, pl.ds(NS * DEG_CHUNK, 16)])

    return _scatter_kernel


_scatter_by_f = {F: _make_scatter(F) for F in (64, 32, 16)}


G_TC = 10
RB = N_NODES_ // G_TC


def _row0_mask():
    ridx = lax.broadcasted_iota(jnp.int32, (RB, 1), 0)
    return (ridx == 0) & (pl.program_id(0) == 0)


def _k1_body(x_ref, w_ref, degp_ref, y_ref, dis_ref):
    pad_deg = jnp.where(_row0_mask(), float(K_PAD), 0.0)
    deg = degp_ref[:, 0:1] + degp_ref[:, 1:2] + 1.0 - pad_deg
    dis = lax.rsqrt(deg)
    h = jnp.dot(x_ref[...], w_ref[...],
                preferred_element_type=jnp.float32,
                precision=lax.Precision.DEFAULT)
    y_ref[...] = h * dis
    dis_ref[...] = dis


def _k1(x, w1, degpT):
    fo = w1.shape[1]
    return pl.pallas_call(
        _k1_body,
        grid=(G_TC,),
        in_specs=[
            pl.BlockSpec((RB, x.shape[1]), lambda i: (i, 0)),
            pl.BlockSpec(w1.shape, lambda i: (0, 0)),
            pl.BlockSpec((RB, 2), lambda i: (i, 0)),
        ],
        out_specs=[
            pl.BlockSpec((RB, fo), lambda i: (i, 0)),
            pl.BlockSpec((RB, 1), lambda i: (i, 0)),
        ],
        out_shape=[
            jax.ShapeDtypeStruct((N_NODES_, fo), jnp.float32),
            jax.ShapeDtypeStruct((N_NODES_, 1), jnp.float32),
        ],
    )(x, w1, degpT)


def _mid_body(p_ref, y_ref, dis_ref, b_ref, w_ref, out_ref):
    dis = dis_ref[...]
    wpad = 1.0 + jnp.where(_row0_mask(), float(K_PAD), 0.0)
    z = p_ref[0] + p_ref[1] - wpad * y_ref[...]
    h = jnp.maximum(z * dis + b_ref[...], 0.0)
    out_ref[...] = jnp.dot(h, w_ref[...],
                           preferred_element_type=jnp.float32,
                           precision=lax.Precision.DEFAULT) * dis


def _k_mid(p, y, dis, b, w):
    fi, fo = w.shape
    return pl.pallas_call(
        _mid_body,
        grid=(G_TC,),
        in_specs=[
            pl.BlockSpec((2, RB, fi), lambda i: (0, i, 0)),
            pl.BlockSpec((RB, fi), lambda i: (i, 0)),
            pl.BlockSpec((RB, 1), lambda i: (i, 0)),
            pl.BlockSpec((1, fi), lambda i: (0, 0)),
            pl.BlockSpec((fi, fo), lambda i: (0, 0)),
        ],
        out_specs=pl.BlockSpec((RB, fo), lambda i: (i, 0)),
        out_shape=jax.ShapeDtypeStruct((N_NODES_, fo), jnp.float32),
    )(p, y, dis, b, w)


def _final_body(p_ref, y_ref, dis_ref, b_ref, wfc_ref, bfc_ref, out_ref):
    dis = dis_ref[...]
    wpad = 1.0 + jnp.where(_row0_mask(), float(K_PAD), 0.0)
    z = p_ref[0] + p_ref[1] - wpad * y_ref[...]
    h = jnp.maximum(z * dis + b_ref[...], 0.0)
    out_ref[...] = jnp.dot(h, wfc_ref[...],
                           preferred_element_type=jnp.float32,
                           precision=lax.Precision.DEFAULT) + bfc_ref[...]


def _k_final(p, y, dis, b, wfc, bfc):
    fi = wfc.shape[0]
    return pl.pallas_call(
        _final_body,
        grid=(G_TC,),
        in_specs=[
            pl.BlockSpec((2, RB, fi), lambda i: (0, i, 0)),
            pl.BlockSpec((RB, fi), lambda i: (i, 0)),
            pl.BlockSpec((RB, 1), lambda i: (i, 0)),
            pl.BlockSpec((1, fi), lambda i: (0, 0)),
            pl.BlockSpec((fi, 1), lambda i: (0, 0)),
            pl.BlockSpec((1, 1), lambda i: (0, 0)),
        ],
        out_specs=pl.BlockSpec((RB, 1), lambda i: (i, 0)),
        out_shape=jax.ShapeDtypeStruct((N_NODES_, 1), jnp.float32),
    )(p, y, dis, b, wfc, bfc)


def kernel(x, edge_index, W1, b1, W2, b2, W3, b3, Wfc, bfc):
    pad = jnp.zeros((2, K_PAD), jnp.int32)
    ei = jnp.concatenate([edge_index, pad], axis=1).reshape(2, NW * NBLK, BLK)

    degp = _deg_kernel(ei).reshape(NC, N_NODES_)
    degpT = degp.T

    y1, dis = _k1(x, W1, degpT)
    p1 = _scatter_by_f[64](y1, ei)
    y2 = _k_mid(p1, y1, dis, b1.reshape(1, -1), W2)
    p2 = _scatter_by_f[32](y2, ei)
    y3 = _k_mid(p2, y2, dis, b2.reshape(1, -1), W3)
    p3 = _scatter_by_f[16](y3, ei)
    out = _k_final(p3, y3, dis, b3.reshape(1, -1), Wfc,
                   bfc.reshape(1, 1))
    return out


# speedup vs baseline: 1.6712x; 1.6712x over previous
import functools

import jax
import jax.numpy as jnp
from jax import lax
from jax.experimental import pallas as pl
from jax.experimental.pallas import tpu as pltpu
from jax.experimental.pallas import tpu_sc as plsc

N_NODES_ = 10000
N_EDGES_ = 320000
NC = 2
NS = 16
NW = NC * NS
BLK = 96
NBLK = 105
EPW = NBLK * BLK
E_PAD = NW * EPW
K_PAD = E_PAD - N_EDGES_
NBUF = 5
NGRP = NBLK // NBUF
DEG_CHUNK = 624
HALF = DEG_CHUNK // 2

_MESH = plsc.VectorSubcoreMesh(
    core_axis_name="c", subcore_axis_name="s", num_cores=NC, num_subcores=NS)


@functools.partial(
    pl.kernel,
    out_type=jax.ShapeDtypeStruct((NC * N_NODES_,), jnp.float32),
    mesh=_MESH,
    scratch_types=[
        pltpu.VMEM((NBLK, BLK), jnp.int32),
        pltpu.VMEM((BLK,), jnp.float32),
        pltpu.VMEM((DEG_CHUNK + 16,), jnp.float32),
        pltpu.VMEM_SHARED((N_NODES_,), jnp.float32),
        pltpu.SemaphoreType.DMA,
    ],
    compiler_params=pltpu.CompilerParams(use_tc_tiling_on_sc=False),
)
def _deg_kernel(ei_hbm, out_hbm, dst_v, ones_v, zero_v, acc, sem):
    cid = lax.axis_index("c")
    sid = lax.axis_index("s")
    wid = sid * NC + cid

    one16 = jnp.full((16,), 1.0, jnp.float32)
    zero16 = jnp.zeros((16,), jnp.float32)
    for i in range(BLK // 16):
        ones_v[pl.ds(i * 16, 16)] = one16
    for i in range((DEG_CHUNK + 16) // 16):
        zero_v[pl.ds(i * 16, 16)] = zero16

    pltpu.sync_copy(ei_hbm.at[1, pl.ds(wid * NBLK, NBLK)], dst_v)

    pltpu.sync_copy(zero_v.at[pl.ds(0, DEG_CHUNK)],
                    acc.at[pl.ds(sid * DEG_CHUNK, DEG_CHUNK)])

    @pl.when(sid == NS - 1)
    def _():
        pltpu.sync_copy(zero_v.at[pl.ds(DEG_CHUNK, 16)],
                        acc.at[pl.ds(NS * DEG_CHUNK, 16)])

    plsc.subcore_barrier()

    def body(i, carry):
        pltpu.async_copy(ones_v, acc.at[dst_v.at[i]], sem, add=True)
        return carry

    lax.fori_loop(0, NBLK, body, 0)

    def drain(i, carry):
        pltpu.make_async_copy(out_hbm.at[pl.ds(0, BLK)], ones_v, sem).wait()
        return carry

    lax.fori_loop(0, NBLK, drain, 0)
    plsc.subcore_barrier()

    obase = cid * N_NODES_
    pltpu.sync_copy(acc.at[pl.ds(sid * DEG_CHUNK, DEG_CHUNK)],
                    zero_v.at[pl.ds(0, DEG_CHUNK)])
    pltpu.sync_copy(zero_v.at[pl.ds(0, DEG_CHUNK)],
                    out_hbm.at[pl.ds(obase + sid * DEG_CHUNK, DEG_CHUNK)])

    @pl.when(sid == NS - 1)
    def _():
        pltpu.sync_copy(acc.at[pl.ds(NS * DEG_CHUNK, 16)],
                        zero_v.at[pl.ds(DEG_CHUNK, 16)])
        pltpu.sync_copy(zero_v.at[pl.ds(DEG_CHUNK, 16)],
                        out_hbm.at[pl.ds(obase + NS * DEG_CHUNK, 16)])


def _make_scatter(F):
    @functools.partial(
        pl.kernel,
        out_type=jax.ShapeDtypeStruct((NC, N_NODES_, F), jnp.float32),
        mesh=_MESH,
        scratch_types=[
            pltpu.VMEM((NBLK, BLK), jnp.int32),
            pltpu.VMEM((NBLK, BLK), jnp.int32),
            pltpu.VMEM((NBUF, BLK, F), jnp.float32),
            pltpu.VMEM((HALF + 16, F), jnp.float32),
            pltpu.VMEM_SHARED((N_NODES_, F), jnp.float32),
            pltpu.SemaphoreType.DMA((NBUF,)),
            pltpu.SemaphoreType.DMA((NBUF,)),
        ],
        compiler_params=pltpu.CompilerParams(use_tc_tiling_on_sc=False),
    )
    def _scatter_kernel(y_hbm, ei_hbm, out_hbm,
                        src_v, dst_v, rows_v, stage_v, acc, gsem, ssem):
        cid = lax.axis_index("c")
        sid = lax.axis_index("s")
        wid = sid * NC + cid

        pltpu.sync_copy(ei_hbm.at[0, pl.ds(wid * NBLK, NBLK)], src_v)
        pltpu.sync_copy(ei_hbm.at[1, pl.ds(wid * NBLK, NBLK)], dst_v)

        for b in range(NBUF):
            pltpu.async_copy(y_hbm.at[src_v.at[b]], rows_v.at[b], gsem.at[b])

        r0 = sid * DEG_CHUNK
        for k in range(2):
            pltpu.sync_copy(y_hbm.at[pl.ds(r0 + k * HALF, HALF)],
                            stage_v.at[pl.ds(0, HALF)])
            pltpu.sync_copy(stage_v.at[pl.ds(0, HALF)],
                            acc.at[pl.ds(r0 + k * HALF, HALF)])

        @pl.when(sid == NS - 1)
        def _():
            pltpu.sync_copy(y_hbm.at[pl.ds(NS * DEG_CHUNK, 16)],
                            stage_v.at[pl.ds(HALF, 16)])
            pltpu.sync_copy(stage_v.at[pl.ds(HALF, 16)],
                            acc.at[pl.ds(NS * DEG_CHUNK, 16)])

        plsc.subcore_barrier()

        def outer(g, carry):
            for b in range(NBUF):
                pltpu.make_async_copy(y_hbm.at[pl.ds(0, BLK)],
                                      rows_v.at[b], gsem.at[b]).wait()
                pltpu.async_copy(rows_v.at[b], acc.at[dst_v.at[g * NBUF + b]],
                                 ssem.at[b], add=True)
            for b in range(NBUF):
                pltpu.make_async_copy(y_hbm.at[pl.ds(0, BLK)],
                                      rows_v.at[b], ssem.at[b]).wait()

                @pl.when(g + 1 < NGRP)
                def _():
                    pltpu.async_copy(y_hbm.at[src_v.at[(g + 1) * NBUF + b]],
                                     rows_v.at[b], gsem.at[b])

            return carry

        lax.fori_loop(0, NGRP, outer, 0)
        plsc.subcore_barrier()

        for k in range(2):
            pltpu.sync_copy(acc.at[pl.ds(r0 + k * HALF, HALF)],
                            stage_v.at[pl.ds(0, HALF)])
            pltpu.sync_copy(stage_v.at[pl.ds(0, HALF)],
                            out_hbm.at[cid, pl.ds(r0 + k * HALF, HALF)])

        @pl.when(sid == NS - 1)
        def _():
            pltpu.sync_copy(acc.at[pl.ds(NS * DEG_CHUNK, 16)],
                            stage_v.at[pl.ds(HALF, 16)])
            pltpu.sync_copy(stage_v.at[pl.ds(HALF, 16)],
                            out_hbm.at[cid, pl.ds(NS * DEG_CHUNK, 16)])

    return _scatter_kernel


_scatter_by_f = {F: _make_scatter(F) for F in (64, 32, 16)}


G_TC = 10
RB = N_NODES_ // G_TC


def _row0_mask():
    ridx = lax.broadcasted_iota(jnp.int32, (RB, 1), 0)
    return (ridx == 0) & (pl.program_id(0) == 0)


def _k1_body(x_ref, w_ref, degp_ref, y_ref, dis_ref):
    pad_deg = jnp.where(_row0_mask(), float(K_PAD), 0.0)
    deg = degp_ref[:, 0:1] + degp_ref[:, 1:2] + 1.0 - pad_deg
    dis = lax.rsqrt(deg)
    h = jnp.dot(x_ref[...], w_ref[...],
                preferred_element_type=jnp.float32,
                precision=lax.Precision.DEFAULT)
    y_ref[...] = h * dis
    dis_ref[...] = dis


def _k1(x, w1, degpT):
    fo = w1.shape[1]
    return pl.pallas_call(
        _k1_body,
        grid=(G_TC,),
        in_specs=[
            pl.BlockSpec((RB, x.shape[1]), lambda i: (i, 0)),
            pl.BlockSpec(w1.shape, lambda i: (0, 0)),
            pl.BlockSpec((RB, 2), lambda i: (i, 0)),
        ],
        out_specs=[
            pl.BlockSpec((RB, fo), lambda i: (i, 0)),
            pl.BlockSpec((RB, 1), lambda i: (i, 0)),
        ],
        out_shape=[
            jax.ShapeDtypeStruct((N_NODES_, fo), jnp.float32),
            jax.ShapeDtypeStruct((N_NODES_, 1), jnp.float32),
        ],
    )(x, w1, degpT)


def _mid_body(p_ref, y_ref, dis_ref, b_ref, w_ref, out_ref):
    dis = dis_ref[...]
    wpad = 1.0 + jnp.where(_row0_mask(), float(K_PAD), 0.0)
    z = p_ref[0] + p_ref[1] - wpad * y_ref[...]
    h = jnp.maximum(z * dis + b_ref[...], 0.0)
    out_ref[...] = jnp.dot(h, w_ref[...],
                           preferred_element_type=jnp.float32,
                           precision=lax.Precision.DEFAULT) * dis


def _k_mid(p, y, dis, b, w):
    fi, fo = w.shape
    return pl.pallas_call(
        _mid_body,
        grid=(G_TC,),
        in_specs=[
            pl.BlockSpec((2, RB, fi), lambda i: (0, i, 0)),
            pl.BlockSpec((RB, fi), lambda i: (i, 0)),
            pl.BlockSpec((RB, 1), lambda i: (i, 0)),
            pl.BlockSpec((1, fi), lambda i: (0, 0)),
            pl.BlockSpec((fi, fo), lambda i: (0, 0)),
        ],
        out_specs=pl.BlockSpec((RB, fo), lambda i: (i, 0)),
        out_shape=jax.ShapeDtypeStruct((N_NODES_, fo), jnp.float32),
    )(p, y, dis, b, w)


def _final_body(p_ref, y_ref, dis_ref, b_ref, wfc_ref, bfc_ref, out_ref):
    dis = dis_ref[...]
    wpad = 1.0 + jnp.where(_row0_mask(), float(K_PAD), 0.0)
    z = p_ref[0] + p_ref[1] - wpad * y_ref[...]
    h = jnp.maximum(z * dis + b_ref[...], 0.0)
    out_ref[...] = jnp.dot(h, wfc_ref[...],
                           preferred_element_type=jnp.float32,
                           precision=lax.Precision.DEFAULT) + bfc_ref[...]


def _k_final(p, y, dis, b, wfc, bfc):
    fi = wfc.shape[0]
    return pl.pallas_call(
        _final_body,
        grid=(G_TC,),
        in_specs=[
            pl.BlockSpec((2, RB, fi), lambda i: (0, i, 0)),
            pl.BlockSpec((RB, fi), lambda i: (i, 0)),
            pl.BlockSpec((RB, 1), lambda i: (i, 0)),
            pl.BlockSpec((1, fi), lambda i: (0, 0)),
            pl.BlockSpec((fi, 1), lambda i: (0, 0)),
            pl.BlockSpec((1, 1), lambda i: (0, 0)),
        ],
        out_specs=pl.BlockSpec((RB, 1), lambda i: (i, 0)),
        out_shape=jax.ShapeDtypeStruct((N_NODES_, 1), jnp.float32),
    )(p, y, dis, b, wfc, bfc)


def kernel(x, edge_index, W1, b1, W2, b2, W3, b3, Wfc, bfc):
    pad = jnp.zeros((2, K_PAD), jnp.int32)
    ei = jnp.concatenate([edge_index, pad], axis=1).reshape(2, NW * NBLK, BLK)

    degp = _deg_kernel(ei).reshape(NC, N_NODES_)
    degpT = degp.T

    y1, dis = _k1(x, W1, degpT)
    p1 = _scatter_by_f[64](y1, ei)
    y2 = _k_mid(p1, y1, dis, b1.reshape(1, -1), W2)
    p2 = _scatter_by_f[32](y2, ei)
    y3 = _k_mid(p2, y2, dis, b2.reshape(1, -1), W3)
    p3 = _scatter_by_f[16](y3, ei)
    out = _k_final(p3, y3, dis, b3.reshape(1, -1), Wfc,
                   bfc.reshape(1, 1))
    return out


# speedup vs baseline: 2.1778x; 1.3031x over previous
import functools

import jax
import jax.numpy as jnp
from jax import lax
from jax.experimental import pallas as pl
from jax.experimental.pallas import tpu as pltpu
from jax.experimental.pallas import tpu_sc as plsc

N_NODES_ = 10000
N_EDGES_ = 320000
NC = 2
NS = 16
NW = NC * NS
BLK = 80
NBLK = 125
EPW = NBLK * BLK
E_PAD = NW * EPW
K_PAD = E_PAD - N_EDGES_
NBUF = 5
NGRP = NBLK // NBUF
DEG_CHUNK = 624
HALF = DEG_CHUNK // 2

_MESH = plsc.VectorSubcoreMesh(
    core_axis_name="c", subcore_axis_name="s", num_cores=NC, num_subcores=NS)


@functools.partial(
    pl.kernel,
    out_type=jax.ShapeDtypeStruct((NC * N_NODES_,), jnp.float32),
    mesh=_MESH,
    scratch_types=[
        pltpu.VMEM((NBLK, BLK), jnp.int32),
        pltpu.VMEM((BLK,), jnp.float32),
        pltpu.VMEM((DEG_CHUNK + 16,), jnp.float32),
        pltpu.VMEM_SHARED((N_NODES_,), jnp.float32),
        pltpu.SemaphoreType.DMA,
    ],
    compiler_params=pltpu.CompilerParams(use_tc_tiling_on_sc=False),
)
def _deg_kernel(ei_hbm, out_hbm, dst_v, ones_v, zero_v, acc, sem):
    cid = lax.axis_index("c")
    sid = lax.axis_index("s")
    wid = sid * NC + cid

    one16 = jnp.full((16,), 1.0, jnp.float32)
    zero16 = jnp.zeros((16,), jnp.float32)
    for i in range(BLK // 16):
        ones_v[pl.ds(i * 16, 16)] = one16
    for i in range((DEG_CHUNK + 16) // 16):
        zero_v[pl.ds(i * 16, 16)] = zero16

    pltpu.sync_copy(ei_hbm.at[1, pl.ds(wid * NBLK, NBLK)], dst_v)

    pltpu.sync_copy(zero_v.at[pl.ds(0, DEG_CHUNK)],
                    acc.at[pl.ds(sid * DEG_CHUNK, DEG_CHUNK)])

    @pl.when(sid == NS - 1)
    def _():
        pltpu.sync_copy(zero_v.at[pl.ds(DEG_CHUNK, 16)],
                        acc.at[pl.ds(NS * DEG_CHUNK, 16)])

    plsc.subcore_barrier()

    def body(i, carry):
        pltpu.async_copy(ones_v, acc.at[dst_v.at[i]], sem, add=True)
        return carry

    lax.fori_loop(0, NBLK, body, 0)

    def drain(i, carry):
        pltpu.make_async_copy(out_hbm.at[pl.ds(0, BLK)], ones_v, sem).wait()
        return carry

    lax.fori_loop(0, NBLK, drain, 0)
    plsc.subcore_barrier()

    obase = cid * N_NODES_
    pltpu.sync_copy(acc.at[pl.ds(sid * DEG_CHUNK, DEG_CHUNK)],
                    zero_v.at[pl.ds(0, DEG_CHUNK)])
    pltpu.sync_copy(zero_v.at[pl.ds(0, DEG_CHUNK)],
                    out_hbm.at[pl.ds(obase + sid * DEG_CHUNK, DEG_CHUNK)])

    @pl.when(sid == NS - 1)
    def _():
        pltpu.sync_copy(acc.at[pl.ds(NS * DEG_CHUNK, 16)],
                        zero_v.at[pl.ds(DEG_CHUNK, 16)])
        pltpu.sync_copy(zero_v.at[pl.ds(DEG_CHUNK, 16)],
                        out_hbm.at[pl.ds(obase + NS * DEG_CHUNK, 16)])


def _make_scatter(F):
    @functools.partial(
        pl.kernel,
        out_type=jax.ShapeDtypeStruct((NC, N_NODES_, F), jnp.float32),
        mesh=_MESH,
        scratch_types=[
            pltpu.VMEM((NBLK, BLK), jnp.int32),
            pltpu.VMEM((NBLK, BLK), jnp.int32),
            pltpu.VMEM((NBUF, BLK, F), jnp.float32),
            pltpu.VMEM((HALF + 16, F), jnp.float32),
            pltpu.VMEM_SHARED((N_NODES_, F), jnp.float32),
            pltpu.SemaphoreType.DMA((NBUF,)),
            pltpu.SemaphoreType.DMA((NBUF,)),
        ],
        compiler_params=pltpu.CompilerParams(use_tc_tiling_on_sc=False),
    )
    def _scatter_kernel(y_hbm, ei_hbm, out_hbm,
                        src_v, dst_v, rows_v, stage_v, acc, gsem, ssem):
        cid = lax.axis_index("c")
        sid = lax.axis_index("s")
        wid = sid * NC + cid

        pltpu.sync_copy(ei_hbm.at[0, pl.ds(wid * NBLK, NBLK)], src_v)
        pltpu.sync_copy(ei_hbm.at[1, pl.ds(wid * NBLK, NBLK)], dst_v)

        for b in range(NBUF):
            pltpu.async_copy(y_hbm.at[src_v.at[b]], rows_v.at[b], gsem.at[b])

        r0 = sid * DEG_CHUNK
        for k in range(2):
            pltpu.sync_copy(y_hbm.at[pl.ds(r0 + k * HALF, HALF)],
                            stage_v.at[pl.ds(0, HALF)])
            pltpu.sync_copy(stage_v.at[pl.ds(0, HALF)],
                            acc.at[pl.ds(r0 + k * HALF, HALF)])

        @pl.when(sid == NS - 1)
        def _():
            pltpu.sync_copy(y_hbm.at[pl.ds(NS * DEG_CHUNK, 16)],
                            stage_v.at[pl.ds(HALF, 16)])
            pltpu.sync_copy(stage_v.at[pl.ds(HALF, 16)],
                            acc.at[pl.ds(NS * DEG_CHUNK, 16)])

        plsc.subcore_barrier()

        def outer(g, carry):
            for b in range(NBUF):
                pltpu.make_async_copy(y_hbm.at[pl.ds(0, BLK)],
                                      rows_v.at[b], gsem.at[b]).wait()
                pltpu.async_copy(rows_v.at[b], acc.at[dst_v.at[g * NBUF + b]],
                                 ssem.at[b], add=True)
            for b in range(NBUF):
                pltpu.make_async_copy(y_hbm.at[pl.ds(0, BLK)],
                                      rows_v.at[b], ssem.at[b]).wait()

                @pl.when(g + 1 < NGRP)
                def _():
                    pltpu.async_copy(y_hbm.at[src_v.at[(g + 1) * NBUF + b]],
                                     rows_v.at[b], gsem.at[b])

            return carry

        lax.fori_loop(0, NGRP, outer, 0)
        plsc.subcore_barrier()

        for k in range(2):
            pltpu.sync_copy(acc.at[pl.ds(r0 + k * HALF, HALF)],
                            stage_v.at[pl.ds(0, HALF)])
            pltpu.sync_copy(stage_v.at[pl.ds(0, HALF)],
                            out_hbm.at[cid, pl.ds(r0 + k * HALF, HALF)])

        @pl.when(sid == NS - 1)
        def _():
            pltpu.sync_copy(acc.at[pl.ds(NS * DEG_CHUNK, 16)],
                            stage_v.at[pl.ds(HALF, 16)])
            pltpu.sync_copy(stage_v.at[pl.ds(HALF, 16)],
                            out_hbm.at[cid, pl.ds(NS * DEG_CHUNK, 16)])

    return _scatter_kernel


_scatter_by_f = {F: _make_scatter(F) for F in (64, 32, 16)}


G_TC = 10
RB = N_NODES_ // G_TC


def _row0_mask():
    ridx = lax.broadcasted_iota(jnp.int32, (RB, 1), 0)
    return (ridx == 0) & (pl.program_id(0) == 0)


def _k1_body(x_ref, w_ref, degp_ref, y_ref, dis_ref):
    pad_deg = jnp.where(_row0_mask(), float(K_PAD), 0.0)
    deg = degp_ref[:, 0:1] + degp_ref[:, 1:2] + 1.0 - pad_deg
    dis = lax.rsqrt(deg)
    h = jnp.dot(x_ref[...], w_ref[...],
                preferred_element_type=jnp.float32,
                precision=lax.Precision.DEFAULT)
    y_ref[...] = h * dis
    dis_ref[...] = dis


def _k1(x, w1, degpT):
    fo = w1.shape[1]
    return pl.pallas_call(
        _k1_body,
        grid=(G_TC,),
        in_specs=[
            pl.BlockSpec((RB, x.shape[1]), lambda i: (i, 0)),
            pl.BlockSpec(w1.shape, lambda i: (0, 0)),
            pl.BlockSpec((RB, 2), lambda i: (i, 0)),
        ],
        out_specs=[
            pl.BlockSpec((RB, fo), lambda i: (i, 0)),
            pl.BlockSpec((RB, 1), lambda i: (i, 0)),
        ],
        out_shape=[
            jax.ShapeDtypeStruct((N_NODES_, fo), jnp.float32),
            jax.ShapeDtypeStruct((N_NODES_, 1), jnp.float32),
        ],
    )(x, w1, degpT)


def _mid_body(p_ref, y_ref, dis_ref, b_ref, w_ref, out_ref):
    dis = dis_ref[...]
    wpad = 1.0 + jnp.where(_row0_mask(), float(K_PAD), 0.0)
    z = p_ref[0] + p_ref[1] - wpad * y_ref[...]
    h = jnp.maximum(z * dis + b_ref[...], 0.0)
    out_ref[...] = jnp.dot(h, w_ref[...],
                           preferred_element_type=jnp.float32,
                           precision=lax.Precision.DEFAULT) * dis


def _k_mid(p, y, dis, b, w):
    fi, fo = w.shape
    return pl.pallas_call(
        _mid_body,
        grid=(G_TC,),
        in_specs=[
            pl.BlockSpec((2, RB, fi), lambda i: (0, i, 0)),
            pl.BlockSpec((RB, fi), lambda i: (i, 0)),
            pl.BlockSpec((RB, 1), lambda i: (i, 0)),
            pl.BlockSpec((1, fi), lambda i: (0, 0)),
            pl.BlockSpec((fi, fo), lambda i: (0, 0)),
        ],
        out_specs=pl.BlockSpec((RB, fo), lambda i: (i, 0)),
        out_shape=jax.ShapeDtypeStruct((N_NODES_, fo), jnp.float32),
    )(p, y, dis, b, w)


def _final_body(p_ref, y_ref, dis_ref, b_ref, wfc_ref, bfc_ref, out_ref):
    dis = dis_ref[...]
    wpad = 1.0 + jnp.where(_row0_mask(), float(K_PAD), 0.0)
    z = p_ref[0] + p_ref[1] - wpad * y_ref[...]
    h = jnp.maximum(z * dis + b_ref[...], 0.0)
    out_ref[...] = jnp.dot(h, wfc_ref[...],
                           preferred_element_type=jnp.float32,
                           precision=lax.Precision.DEFAULT) + bfc_ref[...]


def _k_final(p, y, dis, b, wfc, bfc):
    fi = wfc.shape[0]
    return pl.pallas_call(
        _final_body,
        grid=(G_TC,),
        in_specs=[
            pl.BlockSpec((2, RB, fi), lambda i: (0, i, 0)),
            pl.BlockSpec((RB, fi), lambda i: (i, 0)),
            pl.BlockSpec((RB, 1), lambda i: (i, 0)),
            pl.BlockSpec((1, fi), lambda i: (0, 0)),
            pl.BlockSpec((fi, 1), lambda i: (0, 0)),
            pl.BlockSpec((1, 1), lambda i: (0, 0)),
        ],
        out_specs=pl.BlockSpec((RB, 1), lambda i: (i, 0)),
        out_shape=jax.ShapeDtypeStruct((N_NODES_, 1), jnp.float32),
    )(p, y, dis, b, wfc, bfc)


def kernel(x, edge_index, W1, b1, W2, b2, W3, b3, Wfc, bfc):
    pad = jnp.zeros((2, K_PAD), jnp.int32)
    ei = jnp.concatenate([edge_index, pad], axis=1).reshape(2, NW * NBLK, BLK)

    degp = _deg_kernel(ei).reshape(NC, N_NODES_)
    degpT = degp.T

    y1, dis = _k1(x, W1, degpT)
    p1 = _scatter_by_f[64](y1, ei)
    y2 = _k_mid(p1, y1, dis, b1.reshape(1, -1), W2)
    p2 = _scatter_by_f[32](y2, ei)
    y3 = _k_mid(p2, y2, dis, b2.reshape(1, -1), W3)
    p3 = _scatter_by_f[16](y3, ei)
    out = _k_final(p3, y3, dis, b3.reshape(1, -1), Wfc,
                   bfc.reshape(1, 1))
    return out


# speedup vs baseline: 2.4212x; 1.1118x over previous
import functools

import jax
import jax.numpy as jnp
from jax import lax
from jax.experimental import pallas as pl
from jax.experimental.pallas import tpu as pltpu
from jax.experimental.pallas import tpu_sc as plsc

N_NODES_ = 10000
N_EDGES_ = 320000
NC = 2
NS = 16
NW = NC * NS
BLK = 80
NBLK = 125
EPW = NBLK * BLK
E_PAD = NW * EPW
K_PAD = E_PAD - N_EDGES_
NBUF = 5
NGRP = NBLK // NBUF
DEG_CHUNK = 624
HALF = DEG_CHUNK // 2

_MESH = plsc.VectorSubcoreMesh(
    core_axis_name="c", subcore_axis_name="s", num_cores=NC, num_subcores=NS)


@functools.partial(
    pl.kernel,
    out_type=jax.ShapeDtypeStruct((NC * N_NODES_,), jnp.float32),
    mesh=_MESH,
    scratch_types=[
        pltpu.VMEM((NBLK, BLK), jnp.int32),
        pltpu.VMEM((BLK,), jnp.float32),
        pltpu.VMEM((DEG_CHUNK + 16,), jnp.float32),
        pltpu.VMEM_SHARED((N_NODES_,), jnp.float32),
        pltpu.SemaphoreType.DMA,
    ],
    compiler_params=pltpu.CompilerParams(use_tc_tiling_on_sc=False),
)
def _deg_kernel(ei_hbm, out_hbm, dst_v, ones_v, zero_v, acc, sem):
    cid = lax.axis_index("c")
    sid = lax.axis_index("s")
    wid = sid * NC + cid

    one16 = jnp.full((16,), 1.0, jnp.float32)
    zero16 = jnp.zeros((16,), jnp.float32)
    for i in range(BLK // 16):
        ones_v[pl.ds(i * 16, 16)] = one16
    for i in range((DEG_CHUNK + 16) // 16):
        zero_v[pl.ds(i * 16, 16)] = zero16

    pltpu.sync_copy(ei_hbm.at[1, pl.ds(wid * NBLK, NBLK)], dst_v)

    pltpu.sync_copy(zero_v.at[pl.ds(0, DEG_CHUNK)],
                    acc.at[pl.ds(sid * DEG_CHUNK, DEG_CHUNK)])

    @pl.when(sid == NS - 1)
    def _():
        pltpu.sync_copy(zero_v.at[pl.ds(DEG_CHUNK, 16)],
                        acc.at[pl.ds(NS * DEG_CHUNK, 16)])

    plsc.subcore_barrier()

    def body(i, carry):
        pltpu.async_copy(ones_v, acc.at[dst_v.at[i]], sem, add=True)
        return carry

    lax.fori_loop(0, NBLK, body, 0)

    def drain(i, carry):
        pltpu.make_async_copy(out_hbm.at[pl.ds(0, BLK)], ones_v, sem).wait()
        return carry

    lax.fori_loop(0, NBLK, drain, 0)
    plsc.subcore_barrier()

    obase = cid * N_NODES_
    pltpu.sync_copy(acc.at[pl.ds(sid * DEG_CHUNK, DEG_CHUNK)],
                    zero_v.at[pl.ds(0, DEG_CHUNK)])
    pltpu.sync_copy(zero_v.at[pl.ds(0, DEG_CHUNK)],
                    out_hbm.at[pl.ds(obase + sid * DEG_CHUNK, DEG_CHUNK)])

    @pl.when(sid == NS - 1)
    def _():
        pltpu.sync_copy(acc.at[pl.ds(NS * DEG_CHUNK, 16)],
                        zero_v.at[pl.ds(DEG_CHUNK, 16)])
        pltpu.sync_copy(zero_v.at[pl.ds(DEG_CHUNK, 16)],
                        out_hbm.at[pl.ds(obase + NS * DEG_CHUNK, 16)])


def _make_scatter(F):
    @functools.partial(
        pl.kernel,
        out_type=jax.ShapeDtypeStruct((N_NODES_, 128), jnp.float32),
        mesh=_MESH,
        scratch_types=[
            pltpu.VMEM((NBLK, BLK), jnp.int32),
            pltpu.VMEM((NBLK, BLK), jnp.int32),
            pltpu.VMEM((NBUF, BLK, F), jnp.float32),
            pltpu.VMEM((DEG_CHUNK + 16, F), jnp.float32),
            pltpu.VMEM_SHARED((N_NODES_, F), jnp.float32),
            pltpu.SemaphoreType.DMA((NBUF,)),
            pltpu.SemaphoreType.DMA((NBUF,)),
        ],
        compiler_params=pltpu.CompilerParams(use_tc_tiling_on_sc=False),
    )
    def _scatter_kernel(y_hbm, ei_hbm, out_hbm,
                        src_v, dst_v, rows_v, stage_v, acc, gsem, ssem):
        cid = lax.axis_index("c")
        sid = lax.axis_index("s")
        wid = sid * NC + cid

        pltpu.sync_copy(ei_hbm.at[0, pl.ds(wid * NBLK, NBLK)], src_v)
        pltpu.sync_copy(ei_hbm.at[1, pl.ds(wid * NBLK, NBLK)], dst_v)

        for b in range(NBUF):
            pltpu.async_copy(y_hbm.at[src_v.at[b]], rows_v.at[b], gsem.at[b])

        r0 = sid * DEG_CHUNK
        pltpu.sync_copy(y_hbm.at[pl.ds(r0, DEG_CHUNK)],
                        stage_v.at[pl.ds(0, DEG_CHUNK)])
        pltpu.sync_copy(stage_v.at[pl.ds(0, DEG_CHUNK)],
                        acc.at[pl.ds(r0, DEG_CHUNK)])

        @pl.when(sid == NS - 1)
        def _():
            pltpu.sync_copy(y_hbm.at[pl.ds(NS * DEG_CHUNK, 16)],
                            stage_v.at[pl.ds(DEG_CHUNK, 16)])
            pltpu.sync_copy(stage_v.at[pl.ds(DEG_CHUNK, 16)],
                            acc.at[pl.ds(NS * DEG_CHUNK, 16)])

        plsc.subcore_barrier()

        def outer(g, carry):
            for b in range(NBUF):
                pltpu.make_async_copy(y_hbm.at[pl.ds(0, BLK)],
                                      rows_v.at[b], gsem.at[b]).wait()
                pltpu.async_copy(rows_v.at[b], acc.at[dst_v.at[g * NBUF + b]],
                                 ssem.at[b], add=True)
            for b in range(NBUF):
                pltpu.make_async_copy(y_hbm.at[pl.ds(0, BLK)],
                                      rows_v.at[b], ssem.at[b]).wait()

                @pl.when(g + 1 < NGRP)
                def _():
                    pltpu.async_copy(y_hbm.at[src_v.at[(g + 1) * NBUF + b]],
                                     rows_v.at[b], gsem.at[b])

            return carry

        lax.fori_loop(0, NGRP, outer, 0)
        plsc.subcore_barrier()

        col0 = cid * F
        pltpu.sync_copy(acc.at[pl.ds(r0, DEG_CHUNK)],
                        stage_v.at[pl.ds(0, DEG_CHUNK)])
        pltpu.sync_copy(stage_v.at[pl.ds(0, DEG_CHUNK)],
                        out_hbm.at[pl.ds(r0, DEG_CHUNK), pl.ds(col0, F)])

        @pl.when(sid == NS - 1)
        def _():
            pltpu.sync_copy(acc.at[pl.ds(NS * DEG_CHUNK, 16)],
                            stage_v.at[pl.ds(DEG_CHUNK, 16)])
            pltpu.sync_copy(stage_v.at[pl.ds(DEG_CHUNK, 16)],
                            out_hbm.at[pl.ds(NS * DEG_CHUNK, 16),
                                       pl.ds(col0, F)])

    return _scatter_kernel


_scatter_by_f = {F: _make_scatter(F) for F in (64, 32, 16)}


G_TC = 10
RB = N_NODES_ // G_TC


def _row0_mask():
    ridx = lax.broadcasted_iota(jnp.int32, (RB, 1), 0)
    return (ridx == 0) & (pl.program_id(0) == 0)


def _k1_body(x_ref, w_ref, degp_ref, y_ref, dis_ref):
    pad_deg = jnp.where(_row0_mask(), float(K_PAD), 0.0)
    deg = degp_ref[:, 0:1] + degp_ref[:, 1:2] + 1.0 - pad_deg
    dis = lax.rsqrt(deg)
    h = jnp.dot(x_ref[...], w_ref[...],
                preferred_element_type=jnp.float32,
                precision=lax.Precision.DEFAULT)
    y_ref[...] = h * dis
    dis_ref[...] = dis


def _k1(x, w1, degpT):
    fo = w1.shape[1]
    return pl.pallas_call(
        _k1_body,
        grid=(G_TC,),
        in_specs=[
            pl.BlockSpec((RB, x.shape[1]), lambda i: (i, 0)),
            pl.BlockSpec(w1.shape, lambda i: (0, 0)),
            pl.BlockSpec((RB, 2), lambda i: (i, 0)),
        ],
        out_specs=[
            pl.BlockSpec((RB, fo), lambda i: (i, 0)),
            pl.BlockSpec((RB, 1), lambda i: (i, 0)),
        ],
        out_shape=[
            jax.ShapeDtypeStruct((N_NODES_, fo), jnp.float32),
            jax.ShapeDtypeStruct((N_NODES_, 1), jnp.float32),
        ],
    )(x, w1, degpT)


def _mid_body(fi, p_ref, y_ref, dis_ref, b_ref, w_ref, out_ref):
    dis = dis_ref[...]
    wpad = 1.0 + jnp.where(_row0_mask(), float(K_PAD), 0.0)
    z = p_ref[:, :fi] + p_ref[:, fi:2 * fi] - wpad * y_ref[...]
    h = jnp.maximum(z * dis + b_ref[...], 0.0)
    out_ref[...] = jnp.dot(h, w_ref[...],
                           preferred_element_type=jnp.float32,
                           precision=lax.Precision.DEFAULT) * dis


def _k_mid(p, y, dis, b, w):
    fi, fo = w.shape
    return pl.pallas_call(
        functools.partial(_mid_body, fi),
        grid=(G_TC,),
        in_specs=[
            pl.BlockSpec((RB, 128), lambda i: (i, 0)),
            pl.BlockSpec((RB, fi), lambda i: (i, 0)),
            pl.BlockSpec((RB, 1), lambda i: (i, 0)),
            pl.BlockSpec((1, fi), lambda i: (0, 0)),
            pl.BlockSpec((fi, fo), lambda i: (0, 0)),
        ],
        out_specs=pl.BlockSpec((RB, fo), lambda i: (i, 0)),
        out_shape=jax.ShapeDtypeStruct((N_NODES_, fo), jnp.float32),
    )(p, y, dis, b, w)


def _final_body(fi, p_ref, y_ref, dis_ref, b_ref, wfc_ref, bfc_ref, out_ref):
    dis = dis_ref[...]
    wpad = 1.0 + jnp.where(_row0_mask(), float(K_PAD), 0.0)
    z = p_ref[:, :fi] + p_ref[:, fi:2 * fi] - wpad * y_ref[...]
    h = jnp.maximum(z * dis + b_ref[...], 0.0)
    out_ref[...] = jnp.dot(h, wfc_ref[...],
                           preferred_element_type=jnp.float32,
                           precision=lax.Precision.DEFAULT) + bfc_ref[...]


def _k_final(p, y, dis, b, wfc, bfc):
    fi = wfc.shape[0]
    return pl.pallas_call(
        functools.partial(_final_body, fi),
        grid=(G_TC,),
        in_specs=[
            pl.BlockSpec((RB, 128), lambda i: (i, 0)),
            pl.BlockSpec((RB, fi), lambda i: (i, 0)),
            pl.BlockSpec((RB, 1), lambda i: (i, 0)),
            pl.BlockSpec((1, fi), lambda i: (0, 0)),
            pl.BlockSpec((fi, 1), lambda i: (0, 0)),
            pl.BlockSpec((1, 1), lambda i: (0, 0)),
        ],
        out_specs=pl.BlockSpec((RB, 1), lambda i: (i, 0)),
        out_shape=jax.ShapeDtypeStruct((N_NODES_, 1), jnp.float32),
    )(p, y, dis, b, wfc, bfc)


def kernel(x, edge_index, W1, b1, W2, b2, W3, b3, Wfc, bfc):
    pad = jnp.zeros((2, K_PAD), jnp.int32)
    ei = jnp.concatenate([edge_index, pad], axis=1).reshape(2, NW * NBLK, BLK)

    degp = _deg_kernel(ei).reshape(NC, N_NODES_)
    degpT = degp.T

    y1, dis = _k1(x, W1, degpT)
    p1 = _scatter_by_f[64](y1, ei)
    y2 = _k_mid(p1, y1, dis, b1.reshape(1, -1), W2)
    p2 = _scatter_by_f[32](y2, ei)
    y3 = _k_mid(p2, y2, dis, b2.reshape(1, -1), W3)
    p3 = _scatter_by_f[16](y3, ei)
    out = _k_final(p3, y3, dis, b3.reshape(1, -1), Wfc,
                   bfc.reshape(1, 1))
    return out


# speedup vs baseline: 2.5135x; 1.0381x over previous
import functools

import jax
import jax.numpy as jnp
from jax import lax
from jax.experimental import pallas as pl
from jax.experimental.pallas import tpu as pltpu
from jax.experimental.pallas import tpu_sc as plsc

N_NODES_ = 10000
N_EDGES_ = 320000
NC = 2
NS = 16
NW = NC * NS
BLK = 80
NBLK = 125
EPW = NBLK * BLK
E_PAD = NW * EPW
K_PAD = E_PAD - N_EDGES_
NBUF = 5
NGRP = NBLK // NBUF
DEG_CHUNK = 624
HALF = DEG_CHUNK // 2

_MESH = plsc.VectorSubcoreMesh(
    core_axis_name="c", subcore_axis_name="s", num_cores=NC, num_subcores=NS)


@functools.partial(
    pl.kernel,
    out_type=jax.ShapeDtypeStruct((NC * N_NODES_,), jnp.float32),
    mesh=_MESH,
    scratch_types=[
        pltpu.VMEM((NBLK, BLK), jnp.int32),
        pltpu.VMEM((BLK,), jnp.float32),
        pltpu.VMEM((DEG_CHUNK + 16,), jnp.float32),
        pltpu.VMEM_SHARED((N_NODES_,), jnp.float32),
        pltpu.SemaphoreType.DMA,
    ],
    compiler_params=pltpu.CompilerParams(use_tc_tiling_on_sc=False),
)
def _deg_kernel(ei_hbm, out_hbm, dst_v, ones_v, zero_v, acc, sem):
    cid = lax.axis_index("c")
    sid = lax.axis_index("s")
    wid = sid * NC + cid

    one16 = jnp.full((16,), 1.0, jnp.float32)
    zero16 = jnp.zeros((16,), jnp.float32)
    for i in range(BLK // 16):
        ones_v[pl.ds(i * 16, 16)] = one16
    for i in range((DEG_CHUNK + 16) // 16):
        zero_v[pl.ds(i * 16, 16)] = zero16

    pltpu.sync_copy(ei_hbm.at[1, pl.ds(wid * NBLK, NBLK)], dst_v)

    pltpu.sync_copy(zero_v.at[pl.ds(0, DEG_CHUNK)],
                    acc.at[pl.ds(sid * DEG_CHUNK, DEG_CHUNK)])

    @pl.when(sid == NS - 1)
    def _():
        pltpu.sync_copy(zero_v.at[pl.ds(DEG_CHUNK, 16)],
                        acc.at[pl.ds(NS * DEG_CHUNK, 16)])

    plsc.subcore_barrier()

    def body(i, carry):
        pltpu.async_copy(ones_v, acc.at[dst_v.at[i]], sem, add=True)
        return carry

    lax.fori_loop(0, NBLK, body, 0)

    def drain(i, carry):
        pltpu.make_async_copy(out_hbm.at[pl.ds(0, BLK)], ones_v, sem).wait()
        return carry

    lax.fori_loop(0, NBLK, drain, 0)
    plsc.subcore_barrier()

    obase = cid * N_NODES_
    pltpu.sync_copy(acc.at[pl.ds(sid * DEG_CHUNK, DEG_CHUNK)],
                    zero_v.at[pl.ds(0, DEG_CHUNK)])
    pltpu.sync_copy(zero_v.at[pl.ds(0, DEG_CHUNK)],
                    out_hbm.at[pl.ds(obase + sid * DEG_CHUNK, DEG_CHUNK)])

    @pl.when(sid == NS - 1)
    def _():
        pltpu.sync_copy(acc.at[pl.ds(NS * DEG_CHUNK, 16)],
                        zero_v.at[pl.ds(DEG_CHUNK, 16)])
        pltpu.sync_copy(zero_v.at[pl.ds(DEG_CHUNK, 16)],
                        out_hbm.at[pl.ds(obase + NS * DEG_CHUNK, 16)])


def _make_scatter(F):
    @functools.partial(
        pl.kernel,
        out_type=jax.ShapeDtypeStruct((N_NODES_, 128), jnp.float32),
        mesh=_MESH,
        scratch_types=[
            pltpu.VMEM((NBLK, BLK), jnp.int32),
            pltpu.VMEM((NBLK, BLK), jnp.int32),
            pltpu.VMEM((NBUF, BLK, F), jnp.float32),
            pltpu.VMEM((DEG_CHUNK + 16, F), jnp.float32),
            pltpu.VMEM_SHARED((N_NODES_, F), jnp.float32),
            pltpu.SemaphoreType.DMA((NBUF,)),
            pltpu.SemaphoreType.DMA((NBUF,)),
        ],
        compiler_params=pltpu.CompilerParams(use_tc_tiling_on_sc=False),
    )
    def _scatter_kernel(y_hbm, ei_hbm, out_hbm,
                        src_v, dst_v, rows_v, stage_v, acc, gsem, ssem):
        cid = lax.axis_index("c")
        sid = lax.axis_index("s")
        wid = sid * NC + cid

        pltpu.sync_copy(ei_hbm.at[0, pl.ds(wid * NBLK, NBLK)], src_v)
        pltpu.sync_copy(ei_hbm.at[1, pl.ds(wid * NBLK, NBLK)], dst_v)

        for b in range(NBUF):
            pltpu.async_copy(y_hbm.at[src_v.at[b]], rows_v.at[b], gsem.at[b])

        r0 = sid * DEG_CHUNK
        pltpu.sync_copy(y_hbm.at[pl.ds(r0, DEG_CHUNK)],
                        stage_v.at[pl.ds(0, DEG_CHUNK)])
        pltpu.sync_copy(stage_v.at[pl.ds(0, DEG_CHUNK)],
                        acc.at[pl.ds(r0, DEG_CHUNK)])

        @pl.when(sid == NS - 1)
        def _():
            pltpu.sync_copy(y_hbm.at[pl.ds(NS * DEG_CHUNK, 16)],
                            stage_v.at[pl.ds(DEG_CHUNK, 16)])
            pltpu.sync_copy(stage_v.at[pl.ds(DEG_CHUNK, 16)],
                            acc.at[pl.ds(NS * DEG_CHUNK, 16)])

        plsc.subcore_barrier()

        def outer(g, carry):
            for b in range(NBUF):
                pltpu.make_async_copy(y_hbm.at[pl.ds(0, BLK)],
                                      rows_v.at[b], gsem.at[b]).wait()
                pltpu.async_copy(rows_v.at[b], acc.at[dst_v.at[g * NBUF + b]],
                                 ssem.at[b], add=True)
            for b in range(NBUF):
                pltpu.make_async_copy(y_hbm.at[pl.ds(0, BLK)],
                                      rows_v.at[b], ssem.at[b]).wait()

                @pl.when(g + 1 < NGRP)
                def _():
                    pltpu.async_copy(y_hbm.at[src_v.at[(g + 1) * NBUF + b]],
                                     rows_v.at[b], gsem.at[b])

            return carry

        lax.fori_loop(0, NGRP, outer, 0)
        plsc.subcore_barrier()

        col0 = cid * F
        pltpu.sync_copy(acc.at[pl.ds(r0, DEG_CHUNK)],
                        stage_v.at[pl.ds(0, DEG_CHUNK)])
        pltpu.sync_copy(stage_v.at[pl.ds(0, DEG_CHUNK)],
                        out_hbm.at[pl.ds(r0, DEG_CHUNK), pl.ds(col0, F)])

        @pl.when(sid == NS - 1)
        def _():
            pltpu.sync_copy(acc.at[pl.ds(NS * DEG_CHUNK, 16)],
                            stage_v.at[pl.ds(DEG_CHUNK, 16)])
            pltpu.sync_copy(stage_v.at[pl.ds(DEG_CHUNK, 16)],
                            out_hbm.at[pl.ds(NS * DEG_CHUNK, 16),
                                       pl.ds(col0, F)])

    return _scatter_kernel


_scatter_by_f = {F: _make_scatter(F) for F in (64, 32, 16)}


G_TC = 5
RB = N_NODES_ // G_TC


def _row0_mask():
    ridx = lax.broadcasted_iota(jnp.int32, (RB, 1), 0)
    return (ridx == 0) & (pl.program_id(0) == 0)


def _k1_body(x_ref, w_ref, degp_ref, y_ref, dis_ref):
    pad_deg = jnp.where(_row0_mask(), float(K_PAD), 0.0)
    deg = degp_ref[:, 0:1] + degp_ref[:, 1:2] + 1.0 - pad_deg
    dis = lax.rsqrt(deg)
    h = jnp.dot(x_ref[...], w_ref[...],
                preferred_element_type=jnp.float32,
                precision=lax.Precision.DEFAULT)
    y_ref[...] = h * dis
    dis_ref[...] = dis


def _k1(x, w1, degpT):
    fo = w1.shape[1]
    return pl.pallas_call(
        _k1_body,
        grid=(G_TC,),
        in_specs=[
            pl.BlockSpec((RB, x.shape[1]), lambda i: (i, 0)),
            pl.BlockSpec(w1.shape, lambda i: (0, 0)),
            pl.BlockSpec((RB, 2), lambda i: (i, 0)),
        ],
        out_specs=[
            pl.BlockSpec((RB, fo), lambda i: (i, 0)),
            pl.BlockSpec((RB, 1), lambda i: (i, 0)),
        ],
        out_shape=[
            jax.ShapeDtypeStruct((N_NODES_, fo), jnp.float32),
            jax.ShapeDtypeStruct((N_NODES_, 1), jnp.float32),
        ],
    )(x, w1, degpT)


def _mid_body(fi, p_ref, y_ref, dis_ref, b_ref, w_ref, out_ref):
    dis = dis_ref[...]
    wpad = 1.0 + jnp.where(_row0_mask(), float(K_PAD), 0.0)
    z = p_ref[:, :fi] + p_ref[:, fi:2 * fi] - wpad * y_ref[...]
    h = jnp.maximum(z * dis + b_ref[...], 0.0)
    out_ref[...] = jnp.dot(h, w_ref[...],
                           preferred_element_type=jnp.float32,
                           precision=lax.Precision.DEFAULT) * dis


def _k_mid(p, y, dis, b, w):
    fi, fo = w.shape
    return pl.pallas_call(
        functools.partial(_mid_body, fi),
        grid=(G_TC,),
        in_specs=[
            pl.BlockSpec((RB, 128), lambda i: (i, 0)),
            pl.BlockSpec((RB, fi), lambda i: (i, 0)),
            pl.BlockSpec((RB, 1), lambda i: (i, 0)),
            pl.BlockSpec((1, fi), lambda i: (0, 0)),
            pl.BlockSpec((fi, fo), lambda i: (0, 0)),
        ],
        out_specs=pl.BlockSpec((RB, fo), lambda i: (i, 0)),
        out_shape=jax.ShapeDtypeStruct((N_NODES_, fo), jnp.float32),
    )(p, y, dis, b, w)


def _final_body(fi, p_ref, y_ref, dis_ref, b_ref, wfc_ref, bfc_ref, out_ref):
    dis = dis_ref[...]
    wpad = 1.0 + jnp.where(_row0_mask(), float(K_PAD), 0.0)
    z = p_ref[:, :fi] + p_ref[:, fi:2 * fi] - wpad * y_ref[...]
    h = jnp.maximum(z * dis + b_ref[...], 0.0)
    out_ref[...] = jnp.dot(h, wfc_ref[...],
                           preferred_element_type=jnp.float32,
                           precision=lax.Precision.DEFAULT) + bfc_ref[...]


def _k_final(p, y, dis, b, wfc, bfc):
    fi = wfc.shape[0]
    return pl.pallas_call(
        functools.partial(_final_body, fi),
        grid=(G_TC,),
        in_specs=[
            pl.BlockSpec((RB, 128), lambda i: (i, 0)),
            pl.BlockSpec((RB, fi), lambda i: (i, 0)),
            pl.BlockSpec((RB, 1), lambda i: (i, 0)),
            pl.BlockSpec((1, fi), lambda i: (0, 0)),
            pl.BlockSpec((fi, 1), lambda i: (0, 0)),
            pl.BlockSpec((1, 1), lambda i: (0, 0)),
        ],
        out_specs=pl.BlockSpec((RB, 1), lambda i: (i, 0)),
        out_shape=jax.ShapeDtypeStruct((N_NODES_, 1), jnp.float32),
    )(p, y, dis, b, wfc, bfc)


def kernel(x, edge_index, W1, b1, W2, b2, W3, b3, Wfc, bfc):
    pad = jnp.zeros((2, K_PAD), jnp.int32)
    ei = jnp.concatenate([edge_index, pad], axis=1).reshape(2, NW * NBLK, BLK)

    degp = _deg_kernel(ei).reshape(NC, N_NODES_)
    degpT = degp.T

    y1, dis = _k1(x, W1, degpT)
    p1 = _scatter_by_f[64](y1, ei)
    y2 = _k_mid(p1, y1, dis, b1.reshape(1, -1), W2)
    p2 = _scatter_by_f[32](y2, ei)
    y3 = _k_mid(p2, y2, dis, b2.reshape(1, -1), W3)
    p3 = _scatter_by_f[16](y3, ei)
    out = _k_final(p3, y3, dis, b3.reshape(1, -1), Wfc,
                   bfc.reshape(1, 1))
    return out


# speedup vs baseline: 2.6031x; 1.0356x over previous
import functools

import jax
import jax.numpy as jnp
from jax import lax
from jax.experimental import pallas as pl
from jax.experimental.pallas import tpu as pltpu
from jax.experimental.pallas import tpu_sc as plsc

N_NODES_ = 10000
N_EDGES_ = 320000
NC = 2
NS = 16
NW = NC * NS
BLK = 80
NBLK = 125
EPW = NBLK * BLK
E_PAD = NW * EPW
K_PAD = E_PAD - N_EDGES_
NBUF = 5
NGRP = NBLK // NBUF
DEG_CHUNK = 624
HALF = DEG_CHUNK // 2

_MESH = plsc.VectorSubcoreMesh(
    core_axis_name="c", subcore_axis_name="s", num_cores=NC, num_subcores=NS)


@functools.partial(
    pl.kernel,
    out_type=jax.ShapeDtypeStruct((NC * N_NODES_,), jnp.float32),
    mesh=_MESH,
    scratch_types=[
        pltpu.VMEM((NBLK, BLK), jnp.int32),
        pltpu.VMEM((BLK,), jnp.float32),
        pltpu.VMEM((DEG_CHUNK + 16,), jnp.float32),
        pltpu.VMEM_SHARED((N_NODES_,), jnp.float32),
        pltpu.SemaphoreType.DMA,
    ],
    compiler_params=pltpu.CompilerParams(use_tc_tiling_on_sc=False),
)
def _deg_kernel(ei_hbm, out_hbm, dst_v, ones_v, zero_v, acc, sem):
    cid = lax.axis_index("c")
    sid = lax.axis_index("s")
    wid = sid * NC + cid

    one16 = jnp.full((16,), 1.0, jnp.float32)
    zero16 = jnp.zeros((16,), jnp.float32)
    for i in range(BLK // 16):
        ones_v[pl.ds(i * 16, 16)] = one16
    for i in range((DEG_CHUNK + 16) // 16):
        zero_v[pl.ds(i * 16, 16)] = zero16

    pltpu.sync_copy(ei_hbm.at[1, pl.ds(wid * NBLK, NBLK)], dst_v)

    pltpu.sync_copy(zero_v.at[pl.ds(0, DEG_CHUNK)],
                    acc.at[pl.ds(sid * DEG_CHUNK, DEG_CHUNK)])

    @pl.when(sid == NS - 1)
    def _():
        pltpu.sync_copy(zero_v.at[pl.ds(DEG_CHUNK, 16)],
                        acc.at[pl.ds(NS * DEG_CHUNK, 16)])

    plsc.subcore_barrier()

    def body(i, carry):
        pltpu.async_copy(ones_v, acc.at[dst_v.at[i]], sem, add=True)
        return carry

    lax.fori_loop(0, NBLK, body, 0)

    def drain(i, carry):
        pltpu.make_async_copy(out_hbm.at[pl.ds(0, BLK)], ones_v, sem).wait()
        return carry

    lax.fori_loop(0, NBLK, drain, 0)
    plsc.subcore_barrier()

    obase = cid * N_NODES_
    pltpu.sync_copy(acc.at[pl.ds(sid * DEG_CHUNK, DEG_CHUNK)],
                    zero_v.at[pl.ds(0, DEG_CHUNK)])
    pltpu.sync_copy(zero_v.at[pl.ds(0, DEG_CHUNK)],
                    out_hbm.at[pl.ds(obase + sid * DEG_CHUNK, DEG_CHUNK)])

    @pl.when(sid == NS - 1)
    def _():
        pltpu.sync_copy(acc.at[pl.ds(NS * DEG_CHUNK, 16)],
                        zero_v.at[pl.ds(DEG_CHUNK, 16)])
        pltpu.sync_copy(zero_v.at[pl.ds(DEG_CHUNK, 16)],
                        out_hbm.at[pl.ds(obase + NS * DEG_CHUNK, 16)])


def _make_scatter(F):
    @functools.partial(
        pl.kernel,
        out_type=jax.ShapeDtypeStruct((N_NODES_, 128), jnp.float32),
        mesh=_MESH,
        scratch_types=[
            pltpu.VMEM((NBLK, BLK), jnp.int32),
            pltpu.VMEM((NBLK, BLK), jnp.int32),
            pltpu.VMEM((NBUF, BLK, F), jnp.float32),
            pltpu.VMEM((DEG_CHUNK + 16, F), jnp.float32),
            pltpu.VMEM_SHARED((N_NODES_, F), jnp.float32),
            pltpu.SemaphoreType.DMA((NBUF,)),
            pltpu.SemaphoreType.DMA((NBUF,)),
        ],
        compiler_params=pltpu.CompilerParams(use_tc_tiling_on_sc=False),
    )
    def _scatter_kernel(y_hbm, ei_hbm, out_hbm,
                        src_v, dst_v, rows_v, stage_v, acc, gsem, ssem):
        cid = lax.axis_index("c")
        sid = lax.axis_index("s")
        wid = sid * NC + cid

        pltpu.sync_copy(ei_hbm.at[0, pl.ds(wid * NBLK, NBLK)], src_v)
        pltpu.sync_copy(ei_hbm.at[1, pl.ds(wid * NBLK, NBLK)], dst_v)

        for b in range(NBUF):
            pltpu.async_copy(y_hbm.at[src_v.at[b]], rows_v.at[b], gsem.at[b])

        r0 = sid * DEG_CHUNK
        pltpu.sync_copy(y_hbm.at[pl.ds(r0, DEG_CHUNK)],
                        stage_v.at[pl.ds(0, DEG_CHUNK)])
        pltpu.sync_copy(stage_v.at[pl.ds(0, DEG_CHUNK)],
                        acc.at[pl.ds(r0, DEG_CHUNK)])

        @pl.when(sid == NS - 1)
        def _():
            pltpu.sync_copy(y_hbm.at[pl.ds(NS * DEG_CHUNK, 16)],
                            stage_v.at[pl.ds(DEG_CHUNK, 16)])
            pltpu.sync_copy(stage_v.at[pl.ds(DEG_CHUNK, 16)],
                            acc.at[pl.ds(NS * DEG_CHUNK, 16)])

        plsc.subcore_barrier()

        def outer(g, carry):
            for b in range(NBUF):
                pltpu.make_async_copy(y_hbm.at[pl.ds(0, BLK)],
                                      rows_v.at[b], gsem.at[b]).wait()
                pltpu.async_copy(rows_v.at[b], acc.at[dst_v.at[g * NBUF + b]],
                                 ssem.at[b], add=True)
            for b in range(NBUF):
                pltpu.make_async_copy(y_hbm.at[pl.ds(0, BLK)],
                                      rows_v.at[b], ssem.at[b]).wait()

                @pl.when(g + 1 < NGRP)
                def _():
                    pltpu.async_copy(y_hbm.at[src_v.at[(g + 1) * NBUF + b]],
                                     rows_v.at[b], gsem.at[b])

            return carry

        lax.fori_loop(0, NGRP, outer, 0)
        plsc.subcore_barrier()

        col0 = cid * F
        pltpu.sync_copy(acc.at[pl.ds(r0, DEG_CHUNK)],
                        stage_v.at[pl.ds(0, DEG_CHUNK)])
        pltpu.sync_copy(stage_v.at[pl.ds(0, DEG_CHUNK)],
                        out_hbm.at[pl.ds(r0, DEG_CHUNK), pl.ds(col0, F)])

        @pl.when(sid == NS - 1)
        def _():
            pltpu.sync_copy(acc.at[pl.ds(NS * DEG_CHUNK, 16)],
                            stage_v.at[pl.ds(DEG_CHUNK, 16)])
            pltpu.sync_copy(stage_v.at[pl.ds(DEG_CHUNK, 16)],
                            out_hbm.at[pl.ds(NS * DEG_CHUNK, 16),
                                       pl.ds(col0, F)])

    return _scatter_kernel


_scatter_by_f = {F: _make_scatter(F) for F in (64, 32, 16)}


G_TC = 2
RB = N_NODES_ // G_TC


def _row0_mask():
    ridx = lax.broadcasted_iota(jnp.int32, (RB, 1), 0)
    return (ridx == 0) & (pl.program_id(0) == 0)


def _k1_body(x_ref, w_ref, degp_ref, y_ref, dis_ref):
    pad_deg = jnp.where(_row0_mask(), float(K_PAD), 0.0)
    deg = degp_ref[:, 0:1] + degp_ref[:, 1:2] + 1.0 - pad_deg
    dis = lax.rsqrt(deg)
    h = jnp.dot(x_ref[...], w_ref[...],
                preferred_element_type=jnp.float32,
                precision=lax.Precision.DEFAULT)
    y_ref[...] = h * dis
    dis_ref[...] = dis


def _k1(x, w1, degpT):
    fo = w1.shape[1]
    return pl.pallas_call(
        _k1_body,
        grid=(G_TC,),
        in_specs=[
            pl.BlockSpec((RB, x.shape[1]), lambda i: (i, 0)),
            pl.BlockSpec(w1.shape, lambda i: (0, 0)),
            pl.BlockSpec((RB, 2), lambda i: (i, 0)),
        ],
        out_specs=[
            pl.BlockSpec((RB, fo), lambda i: (i, 0)),
            pl.BlockSpec((RB, 1), lambda i: (i, 0)),
        ],
        out_shape=[
            jax.ShapeDtypeStruct((N_NODES_, fo), jnp.float32),
            jax.ShapeDtypeStruct((N_NODES_, 1), jnp.float32),
        ],
    )(x, w1, degpT)


def _mid_body(fi, p_ref, y_ref, dis_ref, b_ref, w_ref, out_ref):
    dis = dis_ref[...]
    wpad = 1.0 + jnp.where(_row0_mask(), float(K_PAD), 0.0)
    z = p_ref[:, :fi] + p_ref[:, fi:2 * fi] - wpad * y_ref[...]
    h = jnp.maximum(z * dis + b_ref[...], 0.0)
    out_ref[...] = jnp.dot(h, w_ref[...],
                           preferred_element_type=jnp.float32,
                           precision=lax.Precision.DEFAULT) * dis


def _k_mid(p, y, dis, b, w):
    fi, fo = w.shape
    return pl.pallas_call(
        functools.partial(_mid_body, fi),
        grid=(G_TC,),
        in_specs=[
            pl.BlockSpec((RB, 128), lambda i: (i, 0)),
            pl.BlockSpec((RB, fi), lambda i: (i, 0)),
            pl.BlockSpec((RB, 1), lambda i: (i, 0)),
            pl.BlockSpec((1, fi), lambda i: (0, 0)),
            pl.BlockSpec((fi, fo), lambda i: (0, 0)),
        ],
        out_specs=pl.BlockSpec((RB, fo), lambda i: (i, 0)),
        out_shape=jax.ShapeDtypeStruct((N_NODES_, fo), jnp.float32),
    )(p, y, dis, b, w)


def _final_body(fi, p_ref, y_ref, dis_ref, b_ref, wfc_ref, bfc_ref, out_ref):
    dis = dis_ref[...]
    wpad = 1.0 + jnp.where(_row0_mask(), float(K_PAD), 0.0)
    z = p_ref[:, :fi] + p_ref[:, fi:2 * fi] - wpad * y_ref[...]
    h = jnp.maximum(z * dis + b_ref[...], 0.0)
    out_ref[...] = jnp.dot(h, wfc_ref[...],
                           preferred_element_type=jnp.float32,
                           precision=lax.Precision.DEFAULT) + bfc_ref[...]


def _k_final(p, y, dis, b, wfc, bfc):
    fi = wfc.shape[0]
    return pl.pallas_call(
        functools.partial(_final_body, fi),
        grid=(G_TC,),
        in_specs=[
            pl.BlockSpec((RB, 128), lambda i: (i, 0)),
            pl.BlockSpec((RB, fi), lambda i: (i, 0)),
            pl.BlockSpec((RB, 1), lambda i: (i, 0)),
            pl.BlockSpec((1, fi), lambda i: (0, 0)),
            pl.BlockSpec((fi, 1), lambda i: (0, 0)),
            pl.BlockSpec((1, 1), lambda i: (0, 0)),
        ],
        out_specs=pl.BlockSpec((RB, 1), lambda i: (i, 0)),
        out_shape=jax.ShapeDtypeStruct((N_NODES_, 1), jnp.float32),
    )(p, y, dis, b, wfc, bfc)


def kernel(x, edge_index, W1, b1, W2, b2, W3, b3, Wfc, bfc):
    pad = jnp.zeros((2, K_PAD), jnp.int32)
    ei = jnp.concatenate([edge_index, pad], axis=1).reshape(2, NW * NBLK, BLK)

    degp = _deg_kernel(ei).reshape(NC, N_NODES_)
    degpT = degp.T

    y1, dis = _k1(x, W1, degpT)
    p1 = _scatter_by_f[64](y1, ei)
    y2 = _k_mid(p1, y1, dis, b1.reshape(1, -1), W2)
    p2 = _scatter_by_f[32](y2, ei)
    y3 = _k_mid(p2, y2, dis, b2.reshape(1, -1), W3)
    p3 = _scatter_by_f[16](y3, ei)
    out = _k_final(p3, y3, dis, b3.reshape(1, -1), Wfc,
                   bfc.reshape(1, 1))
    return out
